# Initial kernel scaffold; baseline (speedup 1.0000x reference)
import jax
import jax.numpy as jnp
from jax.experimental import pallas as pl


def kernel(meshes, faces, cams, flows, pix_to_face):
    raise NotImplementedError("write your pallas kernel here")


# speedup vs baseline: 15.3474x; 15.3474x over previous
import functools

import jax
import jax.numpy as jnp
from jax import lax
from jax.experimental import pallas as pl
from jax.experimental.pallas import tpu as pltpu
from jax.experimental.pallas import tpu_sc as plsc

B, T, NV, NF, H, W = 4, 8, 6890, 13776, 224, 224
BT = B * T
NVP = 6912
NPIX = BT * H * W
NFACE = BT * NF
VV = BT * NVP
L = 16

PIX_PER_CORE = NPIX // 2
PIX_PER_TILE = PIX_PER_CORE // 16
PIX_CHUNK = 3136
N_PIX_CHUNK = PIX_PER_TILE // PIX_CHUNK
FACE_PER_TILE = NFACE // 16
FACE_CHUNK = 9184
N_FACE_CHUNK = FACE_PER_TILE // FACE_CHUNK
VV_PER_TILE = VV // 16

VC = 3456
FRAME_FLOW = H * W * 2

_MESH = plsc.VectorSubcoreMesh(core_axis_name="c", subcore_axis_name="s")
_PARAMS = pltpu.CompilerParams(needs_layout_passes=False)


def _fill(ref, n, val, dtype):
    def body(i, _):
        ref[pl.ds(i * L, L)] = jnp.full((L,), val, dtype)
        return 0
    lax.fori_loop(0, n // L, body, 0)


_VIS_SCRATCH = [
    pltpu.VMEM((PIX_CHUNK,), jnp.int32),
    pltpu.VMEM((PIX_CHUNK,), jnp.float32),
    pltpu.VMEM((FACE_CHUNK * 3,), jnp.int32),
    pltpu.VMEM((FACE_CHUNK,), jnp.float32),
    pltpu.VMEM((FACE_CHUNK,), jnp.int32),
    pltpu.VMEM((FACE_CHUNK,), jnp.int32),
    pltpu.VMEM((FACE_CHUNK,), jnp.int32),
    pltpu.VMEM((VV_PER_TILE,), jnp.float32),
    pltpu.VMEM_SHARED((NFACE,), jnp.float32),
    pltpu.VMEM_SHARED((VV,), jnp.float32),
]


def _vis_body(p2f, faces, vv_out,
                idx_buf, ones_buf, faces_buf, cnt_buf,
                sidx0_buf, sidx1_buf, sidx2_buf, zero_buf,
                face_sp, vv_sp):
    sidx_bufs = (sidx0_buf, sidx1_buf, sidx2_buf)
    c = lax.axis_index("c")
    s = lax.axis_index("s")
    iota = lax.iota(jnp.int32, L)

    _fill(zero_buf, VV_PER_TILE, 0.0, jnp.float32)
    _fill(ones_buf, PIX_CHUNK, 1.0, jnp.float32)

    fbase = s * FACE_PER_TILE
    pltpu.sync_copy(zero_buf, face_sp.at[pl.ds(fbase, VV_PER_TILE)])
    pltpu.sync_copy(zero_buf.at[pl.ds(0, FACE_PER_TILE - VV_PER_TILE)],
                    face_sp.at[pl.ds(fbase + VV_PER_TILE,
                                     FACE_PER_TILE - VV_PER_TILE)])
    pltpu.sync_copy(zero_buf, vv_sp.at[pl.ds(s * VV_PER_TILE, VV_PER_TILE)])
    plsc.subcore_barrier()

    pbase = c * PIX_PER_CORE + s * PIX_PER_TILE
    def stage_a(k, _):
        pltpu.sync_copy(p2f.at[pl.ds(pbase + k * PIX_CHUNK, PIX_CHUNK)],
                        idx_buf)
        pltpu.sync_copy(ones_buf, face_sp.at[idx_buf], add=True)
        return 0
    lax.fori_loop(0, N_PIX_CHUNK, stage_a, 0)
    plsc.subcore_barrier()

    def stage_b(k, _):
        gbase = fbase + k * FACE_CHUNK
        pltpu.sync_copy(faces.at[pl.ds(gbase * 3, FACE_CHUNK * 3)], faces_buf)
        pltpu.sync_copy(face_sp.at[pl.ds(gbase, FACE_CHUNK)], cnt_buf)

        def inner(j, _):
            p = j * L + iota
            g = gbase + p
            off = lax.div(g, NF) * NVP
            for r in range(3):
                fr = plsc.load_gather(faces_buf, [p * 3 + r])
                sidx_bufs[r][pl.ds(j * L, L)] = fr + off
            return 0
        lax.fori_loop(0, FACE_CHUNK // L, inner, 0)
        for r in range(3):
            pltpu.sync_copy(cnt_buf, vv_sp.at[sidx_bufs[r]], add=True)
        return 0
    lax.fori_loop(0, N_FACE_CHUNK, stage_b, 0)
    plsc.subcore_barrier()

    pltpu.sync_copy(vv_sp.at[pl.ds(s * VV_PER_TILE, VV_PER_TILE)],
                    vv_out.at[c, pl.ds(s * VV_PER_TILE, VV_PER_TILE)])


_vis_kernel = pl.kernel(
    _vis_body,
    mesh=_MESH,
    compiler_params=_PARAMS,
    out_type=jax.ShapeDtypeStruct((2, VV), jnp.float32),
    scratch_types=_VIS_SCRATCH,
)


def _round_half_even(x):
    t = x.astype(jnp.int32)
    fl = t - jnp.where(x < t.astype(jnp.float32), 1, 0)
    r = x - fl.astype(jnp.float32)
    odd = jnp.bitwise_and(fl, 1)
    up = (r > 0.5) | ((r == 0.5) & (odd == 1))
    return fl + jnp.where(up, 1, 0)


_LOSS_SCRATCH = [
    pltpu.VMEM((FRAME_FLOW,), jnp.float32),
    pltpu.VMEM((VC * 2,), jnp.float32),
    pltpu.VMEM((VC * 2,), jnp.float32),
    pltpu.VMEM((VC,), jnp.float32),
    pltpu.VMEM((VC,), jnp.float32),
    pltpu.VMEM((3 * L,), jnp.float32),
    pltpu.VMEM((3 * L,), jnp.float32),
    pltpu.VMEM((L,), jnp.float32),
]


def _loss_body(mesh_xy, cams, flows, vv, out,
                 flow_buf, mc_buf, mp_buf, vv0_buf, vv1_buf,
                 camc_buf, camp_buf, o_buf):
    c = lax.axis_index("c")
    s = lax.axis_index("s")
    n = s * 2 + c
    tau = lax.rem(n, T)
    pn = lax.max(n - 1, 0)
    iota = lax.iota(jnp.int32, L)

    pltpu.sync_copy(flows.at[pl.ds(n * FRAME_FLOW, FRAME_FLOW)], flow_buf)
    pltpu.sync_copy(cams.at[pl.ds(n * 3 * L, 3 * L)], camc_buf)
    pltpu.sync_copy(cams.at[pl.ds(pn * 3 * L, 3 * L)], camp_buf)
    s_c = camc_buf[pl.ds(0, L)]
    txc = camc_buf[pl.ds(L, L)]
    tyc = camc_buf[pl.ds(2 * L, L)]
    s_p = camp_buf[pl.ds(0, L)]
    txp = camp_buf[pl.ds(L, L)]
    typ = camp_buf[pl.ds(2 * L, L)]
    tauf = jnp.where(tau != 0, 1.0, 0.0).astype(jnp.float32)

    def chunk(ci, carry):
        acc_a, acc_c = carry
        vb = ci * VC
        pltpu.sync_copy(mesh_xy.at[pl.ds((n * NVP + vb) * 2, VC * 2)], mc_buf)
        pltpu.sync_copy(mesh_xy.at[pl.ds((pn * NVP + vb) * 2, VC * 2)], mp_buf)
        pltpu.sync_copy(vv.at[0, pl.ds(n * NVP + vb, VC)], vv0_buf)
        pltpu.sync_copy(vv.at[1, pl.ds(n * NVP + vb, VC)], vv1_buf)

        def inner(j, icarry):
            ia, ic = icarry
            p = j * L + iota
            vl = vb + p
            xc = plsc.load_gather(mc_buf, [p * 2])
            yc = plsc.load_gather(mc_buf, [p * 2 + 1])
            xp = plsc.load_gather(mp_buf, [p * 2])
            yp = plsc.load_gather(mp_buf, [p * 2 + 1])
            ppx = s_c * xc + txc
            ppy = s_c * yc + tyc
            qx = s_p * xp + txp
            qy = s_p * yp + typ
            ix = ((ppx + 1.0) * W - 1.0) * 0.5
            iy = ((ppy + 1.0) * H - 1.0) * 0.5
            ixn = _round_half_even(ix)
            iyn = _round_half_even(iy)
            valid = (ixn >= 0) & (ixn < W) & (iyn >= 0) & (iyn < H)
            ixc = jnp.clip(ixn, 0, W - 1)
            iyc = jnp.clip(iyn, 0, H - 1)
            lin = iyc * W + ixc
            fx = plsc.load_gather(flow_buf, [lin * 2])
            fy = plsc.load_gather(flow_buf, [lin * 2 + 1])
            vf = jnp.where(valid, 1.0, 0.0)
            sx = fx * vf
            sy = fy * vf
            vvv = vv0_buf[pl.ds(j * L, L)] + vv1_buf[pl.ds(j * L, L)]
            visb = ((jnp.abs(sx) + jnp.abs(sy)) != 0.0) & (vvv > 0.0) & (vl < NV)
            vis = jnp.where(visb, 1.0, 0.0) * tauf
            px_ = (ppx + 1.0) * (W * 0.5)
            py_ = (ppy + 1.0) * (W * 0.5)
            qx_ = (qx + 1.0) * (W * 0.5)
            qy_ = (qy + 1.0) * (W * 0.5)
            opx = qx_ - px_
            opy = qy_ - py_
            ia = ia + jnp.abs(vis * sx - vis * opx) + jnp.abs(vis * sy - vis * opy)
            ic = ic + vis
            return (ia, ic)

        return lax.fori_loop(0, VC // L, inner, (acc_a, acc_c))

    zero = jnp.zeros((L,), jnp.float32)
    acc_a, acc_c = lax.fori_loop(0, NVP // VC, chunk, (zero, zero))
    a = jnp.sum(acc_a)
    cs = jnp.sum(acc_c)
    av = jnp.full((L,), a, jnp.float32)
    cv = jnp.full((L,), cs, jnp.float32)
    o_buf[...] = av / 224.0 / (cv + 1.0)
    pltpu.sync_copy(o_buf, out.at[n])


_loss_kernel = pl.kernel(
    _loss_body,
    mesh=_MESH,
    compiler_params=_PARAMS,
    out_type=jax.ShapeDtypeStruct((BT, L), jnp.float32),
    scratch_types=_LOSS_SCRATCH,
)


@jax.jit
def _impl(meshes, faces, cams, flows, pix_to_face):
    p2f = pix_to_face.reshape(-1).astype(jnp.int32)
    faces_f = faces.reshape(-1).astype(jnp.int32)
    vvcnt = _vis_kernel(p2f, faces_f)

    mesh_xy = jnp.zeros((BT, NVP, 2), jnp.float32)
    mesh_xy = mesh_xy.at[:, :NV, :].set(meshes.reshape(BT, NV, 3)[:, :, :2])
    cams_rep = jnp.broadcast_to(
        cams.reshape(BT, 3)[:, :, None], (BT, 3, L)).reshape(-1)
    flows_f = flows.reshape(-1)
    out = _loss_kernel(mesh_xy.reshape(-1), cams_rep, flows_f, vvcnt)
    return out[:, 0].sum()


def kernel(meshes, faces, cams, flows, pix_to_face):
    return _impl(meshes, faces, cams, flows, pix_to_face)


# speedup vs baseline: 180.1182x; 11.7360x over previous
import functools

import jax
import jax.numpy as jnp
from jax import lax
from jax.experimental import pallas as pl
from jax.experimental.pallas import tpu as pltpu
from jax.experimental.pallas import tpu_sc as plsc

B, T, NV, NF, H, W = 4, 8, 6890, 13776, 224, 224
BT = B * T
NVP = 6912
NPIX = BT * H * W
NFACE = BT * NF
VV = BT * NVP
L = 16

PIX_PER_CORE = NPIX // 2
PIX_PER_TILE = PIX_PER_CORE // 16
PIX_CHUNK = 3136
N_PIX_CHUNK = PIX_PER_TILE // PIX_CHUNK
FACE_PER_TILE = NFACE // 16
FACE_CHUNK = 9184
N_FACE_CHUNK = FACE_PER_TILE // FACE_CHUNK
VV_PER_TILE = VV // 16

VC = 3456
FRAME_FLOW = H * W * 2

_MESH = plsc.VectorSubcoreMesh(core_axis_name="c", subcore_axis_name="s")
_PARAMS = pltpu.CompilerParams(needs_layout_passes=False)


def _fill(ref, n, val, dtype):
    def body(i, _):
        ref[pl.ds(i * L, L)] = jnp.full((L,), val, dtype)
        return 0
    lax.fori_loop(0, n // L, body, 0)


_VIS_SCRATCH = [
    pltpu.VMEM((PIX_CHUNK,), jnp.int32),
    pltpu.VMEM((PIX_CHUNK,), jnp.float32),
    pltpu.VMEM((NF,), jnp.int32),
    pltpu.VMEM((NF,), jnp.float32),
    pltpu.VMEM((NF,), jnp.int32),
    pltpu.VMEM((VV_PER_TILE,), jnp.float32),
    pltpu.VMEM_SHARED((NFACE,), jnp.float32),
    pltpu.VMEM_SHARED((VV,), jnp.float32),
]


def _vis_body(p2f, faces, vv_out,
                idx_buf, ones_buf, faces_buf, cnt_buf, sidx_buf, zero_buf,
                face_sp, vv_sp):
    c = lax.axis_index("c")
    s = lax.axis_index("s")

    _fill(zero_buf, VV_PER_TILE, 0.0, jnp.float32)
    _fill(ones_buf, PIX_CHUNK, 1.0, jnp.float32)

    fbase = s * FACE_PER_TILE
    pltpu.sync_copy(zero_buf, face_sp.at[pl.ds(fbase, VV_PER_TILE)])
    pltpu.sync_copy(zero_buf.at[pl.ds(0, FACE_PER_TILE - VV_PER_TILE)],
                    face_sp.at[pl.ds(fbase + VV_PER_TILE,
                                     FACE_PER_TILE - VV_PER_TILE)])
    pltpu.sync_copy(zero_buf, vv_sp.at[pl.ds(s * VV_PER_TILE, VV_PER_TILE)])
    plsc.subcore_barrier()

    pbase = c * PIX_PER_CORE + s * PIX_PER_TILE
    def stage_a(k, _):
        pltpu.sync_copy(p2f.at[pl.ds(pbase + k * PIX_CHUNK, PIX_CHUNK)],
                        idx_buf)
        pltpu.sync_copy(ones_buf, face_sp.at[idx_buf], add=True)
        return 0
    lax.fori_loop(0, N_PIX_CHUNK, stage_a, 0)
    plsc.subcore_barrier()

    def stage_b(fi, _):
        nf = s * 2 + fi
        fb = lax.div(nf, T)
        ft = lax.rem(nf, T)
        voff = nf * NVP
        pltpu.sync_copy(face_sp.at[pl.ds(nf * NF, NF)], cnt_buf)
        for r in range(3):
            pltpu.sync_copy(
                faces.at[pl.ds(((fb * 3 + r) * T + ft) * NF, NF)], faces_buf)

            def inner(j, _):
                sl = pl.ds(j * L, L)
                sidx_buf[sl] = faces_buf[sl] + voff
                return 0
            lax.fori_loop(0, NF // L, inner, 0)
            pltpu.sync_copy(cnt_buf, vv_sp.at[sidx_buf], add=True)
        return 0
    lax.fori_loop(0, 2, stage_b, 0)
    plsc.subcore_barrier()

    pltpu.sync_copy(vv_sp.at[pl.ds(s * VV_PER_TILE, VV_PER_TILE)],
                    vv_out.at[c, pl.ds(s * VV_PER_TILE, VV_PER_TILE)])


_vis_kernel = pl.kernel(
    _vis_body,
    mesh=_MESH,
    compiler_params=_PARAMS,
    out_type=jax.ShapeDtypeStruct((2, VV), jnp.float32),
    scratch_types=_VIS_SCRATCH,
)


def _round_half_even(x):
    t = x.astype(jnp.int32)
    fl = t - jnp.where(x < t.astype(jnp.float32), 1, 0)
    r = x - fl.astype(jnp.float32)
    odd = jnp.bitwise_and(fl, 1)
    up = (r > 0.5) | ((r == 0.5) & (odd == 1))
    return fl + jnp.where(up, 1, 0)


_LOSS_SCRATCH = [
    pltpu.VMEM((FRAME_FLOW,), jnp.float32),
    pltpu.VMEM((VC,), jnp.float32),
    pltpu.VMEM((VC,), jnp.float32),
    pltpu.VMEM((VC,), jnp.float32),
    pltpu.VMEM((VC,), jnp.float32),
    pltpu.VMEM((VC,), jnp.float32),
    pltpu.VMEM((VC,), jnp.float32),
    pltpu.VMEM((3 * L,), jnp.float32),
    pltpu.VMEM((3 * L,), jnp.float32),
    pltpu.VMEM((L,), jnp.float32),
]


def _loss_body(mesh_xy, cams, flows, vv, out,
                 flow_buf, mx_buf, my_buf, px_buf, py_buf, vv0_buf, vv1_buf,
                 camc_buf, camp_buf, o_buf):
    c = lax.axis_index("c")
    s = lax.axis_index("s")
    n = s * 2 + c
    tau = lax.rem(n, T)
    pn = lax.max(n - 1, 0)
    nb = lax.div(n, T)
    nt = lax.rem(n, T)
    pb = lax.div(pn, T)
    pt = lax.rem(pn, T)
    iota = lax.iota(jnp.int32, L)

    pltpu.sync_copy(flows.at[pl.ds(n * FRAME_FLOW, FRAME_FLOW)], flow_buf)
    pltpu.sync_copy(cams.at[pl.ds(n * 3 * L, 3 * L)], camc_buf)
    pltpu.sync_copy(cams.at[pl.ds(pn * 3 * L, 3 * L)], camp_buf)
    s_c = camc_buf[pl.ds(0, L)]
    txc = camc_buf[pl.ds(L, L)]
    tyc = camc_buf[pl.ds(2 * L, L)]
    s_p = camp_buf[pl.ds(0, L)]
    txp = camp_buf[pl.ds(L, L)]
    typ = camp_buf[pl.ds(2 * L, L)]
    tauf = jnp.where(tau != 0, 1.0, 0.0).astype(jnp.float32)

    def chunk(ci, carry):
        acc_a, acc_c = carry
        vb = ci * VC
        pltpu.sync_copy(
            mesh_xy.at[pl.ds(((nb * 3 + 0) * T + nt) * NVP + vb, VC)], mx_buf)
        pltpu.sync_copy(
            mesh_xy.at[pl.ds(((nb * 3 + 1) * T + nt) * NVP + vb, VC)], my_buf)
        pltpu.sync_copy(
            mesh_xy.at[pl.ds(((pb * 3 + 0) * T + pt) * NVP + vb, VC)], px_buf)
        pltpu.sync_copy(
            mesh_xy.at[pl.ds(((pb * 3 + 1) * T + pt) * NVP + vb, VC)], py_buf)
        pltpu.sync_copy(vv.at[0, pl.ds(n * NVP + vb, VC)], vv0_buf)
        pltpu.sync_copy(vv.at[1, pl.ds(n * NVP + vb, VC)], vv1_buf)

        def inner(j, icarry):
            ia, ic = icarry
            p = j * L + iota
            vl = vb + p
            sl = pl.ds(j * L, L)
            xc = mx_buf[sl]
            yc = my_buf[sl]
            xp = px_buf[sl]
            yp = py_buf[sl]
            ppx = s_c * xc + txc
            ppy = s_c * yc + tyc
            qx = s_p * xp + txp
            qy = s_p * yp + typ
            ix = ((ppx + 1.0) * W - 1.0) * 0.5
            iy = ((ppy + 1.0) * H - 1.0) * 0.5
            ixn = _round_half_even(ix)
            iyn = _round_half_even(iy)
            valid = (ixn >= 0) & (ixn < W) & (iyn >= 0) & (iyn < H)
            ixc = jnp.clip(ixn, 0, W - 1)
            iyc = jnp.clip(iyn, 0, H - 1)
            lin = iyc * (2 * W) + ixc
            fx = plsc.load_gather(flow_buf, [lin])
            fy = plsc.load_gather(flow_buf, [lin + W])
            vf = jnp.where(valid, 1.0, 0.0)
            sx = fx * vf
            sy = fy * vf
            vvv = vv0_buf[sl] + vv1_buf[sl]
            visb = ((jnp.abs(sx) + jnp.abs(sy)) != 0.0) & (vvv > 0.0) & (vl < NV)
            vis = jnp.where(visb, 1.0, 0.0) * tauf
            px_ = (ppx + 1.0) * (W * 0.5)
            py_ = (ppy + 1.0) * (W * 0.5)
            qx_ = (qx + 1.0) * (W * 0.5)
            qy_ = (qy + 1.0) * (W * 0.5)
            opx = qx_ - px_
            opy = qy_ - py_
            ia = ia + jnp.abs(vis * sx - vis * opx) + jnp.abs(vis * sy - vis * opy)
            ic = ic + vis
            return (ia, ic)

        return lax.fori_loop(0, VC // L, inner, (acc_a, acc_c))

    zero = jnp.zeros((L,), jnp.float32)
    acc_a, acc_c = lax.fori_loop(0, NVP // VC, chunk, (zero, zero))
    a = jnp.sum(acc_a)
    cs = jnp.sum(acc_c)
    av = jnp.full((L,), a, jnp.float32)
    cv = jnp.full((L,), cs, jnp.float32)
    o_buf[...] = av / 224.0 / (cv + 1.0)
    pltpu.sync_copy(o_buf, out.at[n])


_loss_kernel = pl.kernel(
    _loss_body,
    mesh=_MESH,
    compiler_params=_PARAMS,
    out_type=jax.ShapeDtypeStruct((BT, L), jnp.float32),
    scratch_types=_LOSS_SCRATCH,
)


@jax.jit
def _impl(meshes, faces, cams, flows, pix_to_face):
    p2f = pix_to_face.reshape(-1).astype(jnp.int32)
    faces_f = faces.transpose(0, 3, 1, 2).reshape(-1).astype(jnp.int32)
    vvcnt = _vis_kernel(p2f, faces_f)

    mesh_t = jnp.pad(meshes.transpose(0, 3, 1, 2),
                     ((0, 0), (0, 0), (0, 0), (0, NVP - NV)))
    cams_rep = jnp.broadcast_to(
        cams.reshape(BT, 3)[:, :, None], (BT, 3, L)).reshape(-1)
    flows_f = flows.transpose(0, 1, 2, 4, 3).reshape(-1)
    out = _loss_kernel(mesh_t.reshape(-1), cams_rep, flows_f, vvcnt)
    return out[:, 0].sum()


def kernel(meshes, faces, cams, flows, pix_to_face):
    return _impl(meshes, faces, cams, flows, pix_to_face)


# speedup vs baseline: 206.6638x; 1.1474x over previous
import functools

import jax
import jax.numpy as jnp
from jax import lax
from jax.experimental import pallas as pl
from jax.experimental.pallas import tpu as pltpu
from jax.experimental.pallas import tpu_sc as plsc

B, T, NV, NF, H, W = 4, 8, 6890, 13776, 224, 224
BT = B * T
NVP = 6912
NPIX = BT * H * W
NFACE = BT * NF
VV = BT * NVP
L = 16

PIX_PER_CORE = NPIX // 2
PIX_PER_TILE = PIX_PER_CORE // 16
PIX_CHUNK = 12544
N_PIX_CHUNK = PIX_PER_TILE // PIX_CHUNK
FACE_PER_TILE = NFACE // 16
FACE_CHUNK = 9184
N_FACE_CHUNK = FACE_PER_TILE // FACE_CHUNK
VV_PER_TILE = VV // 16

VC = 3456
FRAME_FLOW = H * W * 2

_MESH = plsc.VectorSubcoreMesh(core_axis_name="c", subcore_axis_name="s")
_PARAMS = pltpu.CompilerParams(needs_layout_passes=False)


def _fill(ref, n, val, dtype):
    def body(i, _):
        ref[pl.ds(i * L, L)] = jnp.full((L,), val, dtype)
        return 0
    lax.fori_loop(0, n // L, body, 0)


_VIS_SCRATCH = [
    pltpu.VMEM((PIX_CHUNK,), jnp.int32),
    pltpu.VMEM((PIX_CHUNK,), jnp.float32),
    pltpu.VMEM((NF,), jnp.int32),
    pltpu.VMEM((NF,), jnp.float32),
    pltpu.VMEM((VV_PER_TILE,), jnp.float32),
    pltpu.VMEM_SHARED((NFACE,), jnp.float32),
    pltpu.VMEM_SHARED((VV,), jnp.float32),
]


def _vis_body(p2f, faces, vv_out,
                idx_buf, ones_buf, faces_buf, cnt_buf, zero_buf,
                face_sp, vv_sp):
    c = lax.axis_index("c")
    s = lax.axis_index("s")

    _fill(zero_buf, VV_PER_TILE, 0.0, jnp.float32)
    _fill(ones_buf, PIX_CHUNK, 1.0, jnp.float32)

    fbase = s * FACE_PER_TILE
    pltpu.sync_copy(zero_buf, face_sp.at[pl.ds(fbase, VV_PER_TILE)])
    pltpu.sync_copy(zero_buf.at[pl.ds(0, FACE_PER_TILE - VV_PER_TILE)],
                    face_sp.at[pl.ds(fbase + VV_PER_TILE,
                                     FACE_PER_TILE - VV_PER_TILE)])
    pltpu.sync_copy(zero_buf, vv_sp.at[pl.ds(s * VV_PER_TILE, VV_PER_TILE)])
    plsc.subcore_barrier()

    pbase = c * PIX_PER_CORE + s * PIX_PER_TILE
    def stage_a(k, _):
        pltpu.sync_copy(p2f.at[pl.ds(pbase + k * PIX_CHUNK, PIX_CHUNK)],
                        idx_buf)
        pltpu.sync_copy(ones_buf, face_sp.at[idx_buf])
        return 0
    lax.fori_loop(0, N_PIX_CHUNK, stage_a, 0)
    plsc.subcore_barrier()

    def stage_b(fi, _):
        nf = s * 2 + fi
        fb = lax.div(nf, T)
        ft = lax.rem(nf, T)
        voff = nf * NVP
        pltpu.sync_copy(face_sp.at[pl.ds(nf * NF, NF)], cnt_buf)
        for r in range(3):
            pltpu.sync_copy(
                faces.at[pl.ds(((fb * 3 + r) * T + ft) * NF, NF)], faces_buf)
            pltpu.sync_copy(cnt_buf,
                            vv_sp.at[pl.ds(voff, NVP)].at[faces_buf],
                            add=True)
        return 0
    lax.fori_loop(0, 2, stage_b, 0)
    plsc.subcore_barrier()

    pltpu.sync_copy(vv_sp.at[pl.ds(s * VV_PER_TILE, VV_PER_TILE)],
                    vv_out.at[c, pl.ds(s * VV_PER_TILE, VV_PER_TILE)])


_vis_kernel = pl.kernel(
    _vis_body,
    mesh=_MESH,
    compiler_params=_PARAMS,
    out_type=jax.ShapeDtypeStruct((2, VV), jnp.float32),
    scratch_types=_VIS_SCRATCH,
)


def _round_half_even(x):
    t = x.astype(jnp.int32)
    fl = t - jnp.where(x < t.astype(jnp.float32), 1, 0)
    r = x - fl.astype(jnp.float32)
    odd = jnp.bitwise_and(fl, 1)
    up = (r > 0.5) | ((r == 0.5) & (odd == 1))
    return fl + jnp.where(up, 1, 0)


_LOSS_SCRATCH = [
    pltpu.VMEM((FRAME_FLOW,), jnp.float32),
    pltpu.VMEM((VC,), jnp.float32),
    pltpu.VMEM((VC,), jnp.float32),
    pltpu.VMEM((VC,), jnp.float32),
    pltpu.VMEM((VC,), jnp.float32),
    pltpu.VMEM((VC,), jnp.float32),
    pltpu.VMEM((VC,), jnp.float32),
    pltpu.VMEM((3 * L,), jnp.float32),
    pltpu.VMEM((3 * L,), jnp.float32),
    pltpu.VMEM((L,), jnp.float32),
]


def _loss_body(mesh_xy, cams, flows, vv, out,
                 flow_buf, mx_buf, my_buf, px_buf, py_buf, vv0_buf, vv1_buf,
                 camc_buf, camp_buf, o_buf):
    c = lax.axis_index("c")
    s = lax.axis_index("s")
    n = s * 2 + c
    tau = lax.rem(n, T)
    pn = lax.max(n - 1, 0)
    nb = lax.div(n, T)
    nt = lax.rem(n, T)
    pb = lax.div(pn, T)
    pt = lax.rem(pn, T)
    iota = lax.iota(jnp.int32, L)

    pltpu.sync_copy(flows.at[pl.ds(n * FRAME_FLOW, FRAME_FLOW)], flow_buf)
    pltpu.sync_copy(cams.at[pl.ds(n * 3 * L, 3 * L)], camc_buf)
    pltpu.sync_copy(cams.at[pl.ds(pn * 3 * L, 3 * L)], camp_buf)
    s_c = camc_buf[pl.ds(0, L)]
    txc = camc_buf[pl.ds(L, L)]
    tyc = camc_buf[pl.ds(2 * L, L)]
    s_p = camp_buf[pl.ds(0, L)]
    txp = camp_buf[pl.ds(L, L)]
    typ = camp_buf[pl.ds(2 * L, L)]
    tauf = jnp.where(tau != 0, 1.0, 0.0).astype(jnp.float32)

    def chunk(ci, carry):
        acc_a, acc_c = carry
        vb = ci * VC
        pltpu.sync_copy(
            mesh_xy.at[pl.ds(((nb * 3 + 0) * T + nt) * NVP + vb, VC)], mx_buf)
        pltpu.sync_copy(
            mesh_xy.at[pl.ds(((nb * 3 + 1) * T + nt) * NVP + vb, VC)], my_buf)
        pltpu.sync_copy(
            mesh_xy.at[pl.ds(((pb * 3 + 0) * T + pt) * NVP + vb, VC)], px_buf)
        pltpu.sync_copy(
            mesh_xy.at[pl.ds(((pb * 3 + 1) * T + pt) * NVP + vb, VC)], py_buf)
        pltpu.sync_copy(vv.at[0, pl.ds(n * NVP + vb, VC)], vv0_buf)
        pltpu.sync_copy(vv.at[1, pl.ds(n * NVP + vb, VC)], vv1_buf)

        def inner(j, icarry):
            ia, ic = icarry
            p = j * L + iota
            vl = vb + p
            sl = pl.ds(j * L, L)
            xc = mx_buf[sl]
            yc = my_buf[sl]
            xp = px_buf[sl]
            yp = py_buf[sl]
            ppx = s_c * xc + txc
            ppy = s_c * yc + tyc
            qx = s_p * xp + txp
            qy = s_p * yp + typ
            ix = ((ppx + 1.0) * W - 1.0) * 0.5
            iy = ((ppy + 1.0) * H - 1.0) * 0.5
            ixn = _round_half_even(ix)
            iyn = _round_half_even(iy)
            valid = (ixn >= 0) & (ixn < W) & (iyn >= 0) & (iyn < H)
            ixc = jnp.clip(ixn, 0, W - 1)
            iyc = jnp.clip(iyn, 0, H - 1)
            lin = iyc * (2 * W) + ixc
            fx = plsc.load_gather(flow_buf, [lin])
            fy = plsc.load_gather(flow_buf, [lin + W])
            vf = jnp.where(valid, 1.0, 0.0)
            sx = fx * vf
            sy = fy * vf
            vvv = vv0_buf[sl] + vv1_buf[sl]
            visb = ((jnp.abs(sx) + jnp.abs(sy)) != 0.0) & (vvv > 0.0) & (vl < NV)
            vis = jnp.where(visb, 1.0, 0.0) * tauf
            px_ = (ppx + 1.0) * (W * 0.5)
            py_ = (ppy + 1.0) * (W * 0.5)
            qx_ = (qx + 1.0) * (W * 0.5)
            qy_ = (qy + 1.0) * (W * 0.5)
            opx = qx_ - px_
            opy = qy_ - py_
            ia = ia + jnp.abs(vis * sx - vis * opx) + jnp.abs(vis * sy - vis * opy)
            ic = ic + vis
            return (ia, ic)

        return lax.fori_loop(0, VC // L, inner, (acc_a, acc_c))

    zero = jnp.zeros((L,), jnp.float32)
    acc_a, acc_c = lax.fori_loop(0, NVP // VC, chunk, (zero, zero))
    a = jnp.sum(acc_a)
    cs = jnp.sum(acc_c)
    av = jnp.full((L,), a, jnp.float32)
    cv = jnp.full((L,), cs, jnp.float32)
    o_buf[...] = av / 224.0 / (cv + 1.0)
    pltpu.sync_copy(o_buf, out.at[n])


_loss_kernel = pl.kernel(
    _loss_body,
    mesh=_MESH,
    compiler_params=_PARAMS,
    out_type=jax.ShapeDtypeStruct((BT, L), jnp.float32),
    scratch_types=_LOSS_SCRATCH,
)


@jax.jit
def _impl(meshes, faces, cams, flows, pix_to_face):
    p2f = pix_to_face.reshape(-1).astype(jnp.int32)
    faces_f = faces.transpose(0, 3, 1, 2).reshape(-1).astype(jnp.int32)
    vvcnt = _vis_kernel(p2f, faces_f)

    mesh_t = jnp.pad(meshes.transpose(0, 3, 1, 2),
                     ((0, 0), (0, 0), (0, 0), (0, NVP - NV)))
    cams_rep = jnp.broadcast_to(
        cams.reshape(BT, 3)[:, :, None], (BT, 3, L)).reshape(-1)
    flows_f = flows.transpose(0, 1, 2, 4, 3).reshape(-1)
    out = _loss_kernel(mesh_t.reshape(-1), cams_rep, flows_f, vvcnt)
    return out[:, 0].sum()


def kernel(meshes, faces, cams, flows, pix_to_face):
    return _impl(meshes, faces, cams, flows, pix_to_face)
